# baseline (speedup 1.0000x reference)
import functools

import jax
import jax.numpy as jnp
from jax.experimental import pallas as pl
from jax.experimental.pallas import tpu as pltpu

_EPS = 1e-5
_CC = 0.25
_LZ = 10.0
_DV1 = 1.0
_DV2 = 1.0


def _bn_mish(h, g, beta):
    m = jnp.mean(h, axis=0, keepdims=True)
    v = jnp.mean((h - m) ** 2, axis=0, keepdims=True)
    h = (h - m) / jnp.sqrt(v + _EPS) * g + beta
    return h * jnp.tanh(jnp.logaddexp(h, 0.0))


def _layer_body(x_ref, w_ref, b_ref, g_ref, bt_ref, o_ref, acc_ref, *, nk, act):
    k = pl.program_id(0)

    @pl.when(k == 0)
    def _init():
        acc_ref[...] = jnp.zeros_like(acc_ref)

    acc_ref[...] += jax.lax.dot_general(
        x_ref[...], w_ref[...], (((1,), (1,)), ((), ())),
        preferred_element_type=jnp.float32)

    @pl.when(k == nk - 1)
    def _finish():
        h = acc_ref[...] + b_ref[...]
        if act:
            h = _bn_mish(h, g_ref[...], bt_ref[...])
        o_ref[...] = h


def _layer(x, W, b, g, beta, act, k_blk=None):
    B, K = x.shape
    N = W.shape[0]
    if k_blk is None or k_blk > K:
        k_blk = K
    nk = K // k_blk
    if g is None:
        g = jnp.zeros((N,), jnp.float32)
        beta = jnp.zeros((N,), jnp.float32)
    body = functools.partial(_layer_body, nk=nk, act=act)
    return pl.pallas_call(
        body,
        grid=(nk,),
        in_specs=[
            pl.BlockSpec((B, k_blk), lambda k: (0, k)),
            pl.BlockSpec((N, k_blk), lambda k: (0, k)),
            pl.BlockSpec((1, N), lambda k: (0, 0)),
            pl.BlockSpec((1, N), lambda k: (0, 0)),
            pl.BlockSpec((1, N), lambda k: (0, 0)),
        ],
        out_specs=pl.BlockSpec((B, N), lambda k: (0, 0)),
        out_shape=jax.ShapeDtypeStruct((B, N), jnp.float32),
        scratch_shapes=[pltpu.VMEM((B, N), jnp.float32)],
    )(x, W, b.reshape(1, N), g.reshape(1, N), beta.reshape(1, N))


def _final_body(x_ref, w_ref, b_ref, t_ref, o_ref):
    n = pl.program_id(0)
    h = jax.lax.dot_general(
        x_ref[...], w_ref[...], (((1,), (1,)), ((), ())),
        preferred_element_type=jnp.float32)
    d = (h + b_ref[...]) - t_ref[...]

    @pl.when(n == 0)
    def _init():
        o_ref[...] = jnp.zeros_like(o_ref)

    o_ref[...] += jnp.sum(d * d).reshape(1, 1)


def _final_layer_sse(x, W, b, target, n_blk=512):
    B, K = x.shape
    N = W.shape[0]
    nn = N // n_blk
    return pl.pallas_call(
        _final_body,
        grid=(nn,),
        in_specs=[
            pl.BlockSpec((B, K), lambda n: (0, 0)),
            pl.BlockSpec((n_blk, K), lambda n: (n, 0)),
            pl.BlockSpec((1, n_blk), lambda n: (0, n)),
            pl.BlockSpec((B, n_blk), lambda n: (0, n)),
        ],
        out_specs=pl.BlockSpec((1, 1), lambda n: (0, 0)),
        out_shape=jax.ShapeDtypeStruct((1, 1), jnp.float32),
    )(x, W, b.reshape(1, N), target)


def _vq_body(z_ref, z1_ref, cbx_ref, cby_ref, q_ref, q1_ref, s_ref):
    def one(z, cb):
        zz = jnp.sum(z * z, axis=1, keepdims=True)
        cc = jnp.sum(cb * cb, axis=1)[None, :]
        zc = jax.lax.dot_general(
            z, cb, (((1,), (1,)), ((), ())), preferred_element_type=jnp.float32)
        d = zz + cc - 2.0 * zc
        dmin = jnp.min(d, axis=1, keepdims=True)
        ids = jax.lax.broadcasted_iota(jnp.int32, d.shape, 1)
        cand = jnp.where(d <= dmin, ids, d.shape[1])
        idx = jnp.min(cand, axis=1, keepdims=True)
        oh = (ids == idx).astype(jnp.float32)
        q = jax.lax.dot_general(
            oh, cb, (((1,), (0,)), ((), ())), preferred_element_type=jnp.float32)
        sse = jnp.sum((q - z) ** 2)
        return q, sse

    z = z_ref[...]
    z1 = z1_ref[...]
    q, sse_x = one(z, cbx_ref[...])
    q1, sse_y = one(z1, cby_ref[...])
    q_ref[...] = q
    q1_ref[...] = q1
    denom = z.shape[0] * z.shape[1]
    s_ref[...] = (((1.0 + _CC) * (sse_x + sse_y)
                   + _LZ * jnp.sum((z - z1) ** 2)) / denom).reshape(1, 1)


def _vq_both(z, z1, cb_x, cb_y):
    B, E = z.shape
    return pl.pallas_call(
        _vq_body,
        out_shape=(
            jax.ShapeDtypeStruct((B, E), jnp.float32),
            jax.ShapeDtypeStruct((B, E), jnp.float32),
            jax.ShapeDtypeStruct((1, 1), jnp.float32),
        ),
    )(z, z1, cb_x, cb_y)


def _encoder(inp, p):
    h = _layer(inp, p["W"][0], p["b"][0], p["g"][0], p["beta"][0], True, k_blk=512)
    h = _layer(h, p["W"][1], p["b"][1], p["g"][1], p["beta"][1], True)
    h = _layer(h, p["W"][2], p["b"][2], p["g"][2], p["beta"][2], True)
    h = _layer(h, p["W"][3], p["b"][3], None, None, False)
    return h


def _decoder_sse(q, p, target):
    h = _layer(q, p["W"][0], p["b"][0], p["g"][0], p["beta"][0], True)
    h = _layer(h, p["W"][1], p["b"][1], p["g"][1], p["beta"][1], True)
    h = _layer(h, p["W"][2], p["b"][2], p["g"][2], p["beta"][2], True)
    return _final_layer_sse(h, p["W"][3], p["b"][3], target)


def kernel(x, y, params):
    B, in_dim = x.shape
    z = _encoder(x, params["enc_x"])
    z1 = _encoder(y, params["enc_y"])
    q, q1, s_vq = _vq_both(z, z1, params["cb_x"], params["cb_y"])
    sse_x = _decoder_sse(q, params["dec"], y)
    sse_y = _decoder_sse(q1, params["dec"], x)
    recon = (sse_x[0, 0] / _DV1 + sse_y[0, 0] / _DV2) / (B * in_dim)
    return s_vq[0, 0] + recon
